# baseline (speedup 1.0000x reference)
import functools

import jax
import jax.numpy as jnp
from jax import lax
from jax.experimental import pallas as pl
from jax.experimental.pallas import tpu as pltpu
from jax.experimental.pallas import tpu_sc as plsc

N = 50000
E = 800000
G = 512
H = 32

BN = 2000
NB = N // BN
F_DIM = 40

NPAD = 50176
ROWS_PT = NPAD // 16
CH = 128
CPT = 196
RTOT = 32 * CPT
NSLOT = 3
CPS = 14
NSUP = CPT // CPS


def _init_body(h_ref, p_ref, te_ref, gid_ref, f_ref, sums_ref):
    i = pl.program_id(0)
    h = h_ref[...]
    p = p_ref[...]
    te = te_ref[...]
    best = p[:, 0:1]
    am = jnp.zeros((BN, 1), jnp.float32)
    for j in range(1, 4):
        pj = p[:, j:j + 1]
        hit = pj > best
        best = jnp.where(hit, pj, best)
        am = jnp.where(hit, jnp.float32(j), am)
    ones = jnp.ones((BN, 1), jnp.float32)
    zeros = jnp.zeros((BN, 1), jnp.float32)
    F = jnp.concatenate([h, am, te, ones, zeros], axis=1)
    f_ref[...] = F
    gid = jnp.squeeze(gid_ref[...], 0)
    onehot_t = (gid == lax.broadcasted_iota(jnp.int32, (G, BN), 0))
    contrib = jnp.dot(onehot_t.astype(jnp.float32), F,
                      preferred_element_type=jnp.float32)

    @pl.when(i == 0)
    def _():
        sums_ref[...] = contrib

    @pl.when(i > 0)
    def _():
        sums_ref[...] += contrib


def _prep_body(f_ref, sums_ref, gidc_ref, wfs_ref, wfd_ref, wm_ref,
               ps_ref, pd_ref):
    F = f_ref[...]
    sums = sums_ref[...]
    mean = sums[:, :32] / jnp.maximum(sums[:, 38:39], 1.0)
    Mg = jnp.dot(mean, wm_ref[...], preferred_element_type=jnp.float32)
    gidc = gidc_ref[...]
    onehot = (gidc == lax.broadcasted_iota(jnp.int32, (BN, G), 1))
    ps_ref[...] = jnp.dot(F, wfs_ref[...], preferred_element_type=jnp.float32)
    pd_ref[...] = (jnp.dot(F, wfd_ref[...], preferred_element_type=jnp.float32)
                   + jnp.dot(onehot.astype(jnp.float32), Mg,
                             preferred_element_type=jnp.float32))


def _upd_body(f_ref, agg_ref, gid_ref, df_ref, d2_ref, fn_ref, sums_ref):
    i = pl.program_id(0)
    F = f_ref[...]
    a = agg_ref[...]
    agg = a[0] + a[1]
    hn = jnp.maximum(
        jnp.dot(F, df_ref[...], preferred_element_type=jnp.float32)
        + jnp.dot(agg, d2_ref[...], preferred_element_type=jnp.float32), 0.0)
    Fn = jnp.concatenate([hn, F[:, 32:40]], axis=1)
    fn_ref[...] = Fn
    gid = jnp.squeeze(gid_ref[...], 0)
    onehot_t = (gid == lax.broadcasted_iota(jnp.int32, (G, BN), 0))
    contrib = jnp.dot(onehot_t.astype(jnp.float32), Fn,
                      preferred_element_type=jnp.float32)

    @pl.when(i == 0)
    def _():
        sums_ref[...] = contrib

    @pl.when(i > 0)
    def _():
        sums_ref[...] += contrib


def _fin_body(sums_ref, jet_ref, wc0_ref, bc0_ref, wc1_ref, bc1_ref,
              wc2_ref, bc2_ref, out_ref):
    sums = sums_ref[...]
    mean = sums[:, :32] / jnp.maximum(sums[:, 38:39], 1.0)
    gr = jnp.concatenate([mean, jet_ref[...]], axis=1)
    x = jnp.dot(gr, wc0_ref[...], preferred_element_type=jnp.float32) + bc0_ref[...]
    x = jnp.maximum(
        jnp.dot(x, wc1_ref[...], preferred_element_type=jnp.float32)
        + bc1_ref[...], 0.0)
    out_ref[...] = (jnp.dot(x, wc2_ref[...], preferred_element_type=jnp.float32)
                    + bc2_ref[...])


def _node_spec(w):
    return pl.BlockSpec((BN, w), lambda i: (i, 0))


def _full_spec(shape):
    nd = len(shape)
    return pl.BlockSpec(shape, lambda i: (0,) * nd)


def _init_call(node_h, node_pred, node_te, gid3):
    return pl.pallas_call(
        _init_body,
        grid=(NB,),
        in_specs=[_node_spec(32), _node_spec(4), _node_spec(5),
                  pl.BlockSpec((1, 1, BN), lambda i: (i, 0, 0))],
        out_specs=[_node_spec(F_DIM), _full_spec((G, F_DIM))],
        out_shape=[jax.ShapeDtypeStruct((N, F_DIM), jnp.float32),
                   jax.ShapeDtypeStruct((G, F_DIM), jnp.float32)],
    )(node_h, node_pred, node_te, gid3)


def _prep_call(F, sums, gidc, wfs, wfd, wm):
    return pl.pallas_call(
        _prep_body,
        grid=(NB,),
        in_specs=[_node_spec(F_DIM), _full_spec((G, F_DIM)), _node_spec(1),
                  _full_spec((F_DIM, 32)), _full_spec((F_DIM, 32)),
                  _full_spec((32, 32))],
        out_specs=[_node_spec(32), _node_spec(32)],
        out_shape=[jax.ShapeDtypeStruct((N, 32), jnp.float32),
                   jax.ShapeDtypeStruct((N, 32), jnp.float32)],
    )(F, sums, gidc, wfs, wfd, wm)


def _upd_call(F, aggp, gid3, df, d2):
    return pl.pallas_call(
        _upd_body,
        grid=(NB,),
        in_specs=[_node_spec(F_DIM),
                  pl.BlockSpec((2, BN, 32), lambda i: (0, i, 0)),
                  pl.BlockSpec((1, 1, BN), lambda i: (i, 0, 0)),
                  _full_spec((F_DIM, 32)), _full_spec((32, 32))],
        out_specs=[_node_spec(F_DIM), _full_spec((G, F_DIM))],
        out_shape=[jax.ShapeDtypeStruct((N, F_DIM), jnp.float32),
                   jax.ShapeDtypeStruct((G, F_DIM), jnp.float32)],
    )(F, aggp, gid3, df, d2)


def _fin_call(sums, jet, wc0, bc0, wc1, bc1, wc2, bc2):
    return pl.pallas_call(
        _fin_body,
        grid=(1,),
        in_specs=[_full_spec((G, F_DIM)), _full_spec((G, 10)),
                  _full_spec((42, 64)), _full_spec((1, 64)),
                  _full_spec((64, 64)), _full_spec((1, 64)),
                  _full_spec((64, 2)), _full_spec((1, 2))],
        out_specs=_full_spec((G, 2)),
        out_shape=jax.ShapeDtypeStruct((G, 2), jnp.float32),
    )(sums, jet, wc0, bc0, wc1, bc1, wc2, bc2)


def _edge_body(ps_hbm, pd_hbm, src_hbm, dst_hbm, ep_hbm, w0_hbm, zeros_hbm,
               out_hbm, sidx, didx, epv, tv, av3, bv3, w0v, aggsh, gsems,
               ssems):
    c_ax = lax.axis_index("c")
    s_ax = lax.axis_index("s")
    pltpu.sync_copy(zeros_hbm, aggsh.at[pl.ds(s_ax * ROWS_PT, ROWS_PT)])
    pltpu.sync_copy(w0_hbm, w0v)
    plsc.subcore_barrier()
    tile = c_ax * 16 + s_ax

    def super_body(sb, carry0):
        r0 = tile * CPT + sb * CPS
        pltpu.sync_copy(src_hbm.at[pl.ds(r0, CPS)], sidx)
        pltpu.sync_copy(dst_hbm.at[pl.ds(r0, CPS)], didx)
        pltpu.sync_copy(ep_hbm.at[pl.ds(r0, CPS)], epv)
        for k in range(NSLOT - 1):
            pltpu.async_copy(ps_hbm.at[sidx.at[k]], av3.at[k], gsems.at[k])
            pltpu.async_copy(pd_hbm.at[didx.at[k]], bv3.at[k], gsems.at[k])

        def chunk_body(c, carry):
            slot = lax.rem(c, NSLOT)
            p = c + NSLOT - 1
            psl = lax.rem(p, NSLOT)

            @pl.when(p < CPS)
            def _():
                @pl.when(p >= NSLOT)
                def _():
                    pltpu.make_async_copy(av3.at[psl],
                                          aggsh.at[didx.at[p - NSLOT]],
                                          ssems.at[psl]).wait()

                pltpu.async_copy(ps_hbm.at[sidx.at[p]], av3.at[psl],
                                 gsems.at[psl])
                pltpu.async_copy(pd_hbm.at[didx.at[p]], bv3.at[psl],
                                 gsems.at[psl])

            pltpu.make_async_copy(ps_hbm.at[sidx.at[c]], av3.at[slot],
                                  gsems.at[slot]).wait()
            pltpu.make_async_copy(pd_hbm.at[didx.at[c]], bv3.at[slot],
                                  gsems.at[slot]).wait()
            avs = av3.at[slot]
            bvs = bv3.at[slot]

            def sig_body(j, carry2):
                x = epv[c, pl.ds(j * 16, 16)]
                tv[pl.ds(j * 16, 16)] = 1.0 / (1.0 + jnp.exp(-x))
                return carry2

            lax.fori_loop(0, CH // 16, sig_body, 0)
            w0lo = w0v[pl.ds(0, 16)]
            w0hi = w0v[pl.ds(16, 16)]

            def group_body(g, carry3):
                tvec = tv[pl.ds(g * 16, 16)]
                for j in range(16):
                    e = g * 16 + j
                    t = tvec[j]
                    x0 = avs[e, pl.ds(0, 16)] + bvs[e, pl.ds(0, 16)] + t * w0lo
                    x1 = avs[e, pl.ds(16, 16)] + bvs[e, pl.ds(16, 16)] + t * w0hi
                    avs[e, pl.ds(0, 16)] = 1.0 - 2.0 / (jnp.exp(x0 + x0) + 1.0)
                    avs[e, pl.ds(16, 16)] = 1.0 - 2.0 / (jnp.exp(x1 + x1) + 1.0)
                return carry3

            lax.fori_loop(0, CH // 16, group_body, 0)
            pltpu.async_copy(av3.at[slot], aggsh.at[didx.at[c]],
                             ssems.at[slot], add=True)
            return carry

        lax.fori_loop(0, CPS, chunk_body, 0)
        for k in range(NSLOT):
            cc = CPS - NSLOT + k
            pltpu.make_async_copy(av3.at[cc % NSLOT], aggsh.at[didx.at[cc]],
                                  ssems.at[cc % NSLOT]).wait()
        return carry0

    lax.fori_loop(0, NSUP, super_body, 0)
    plsc.subcore_barrier()
    pltpu.sync_copy(aggsh.at[pl.ds(s_ax * ROWS_PT, ROWS_PT)],
                    out_hbm.at[c_ax, pl.ds(s_ax * ROWS_PT, ROWS_PT)])


def _edge_stage(ps, pd, src2d, dst2d, ep2d, w0, zeros):
    mesh = plsc.VectorSubcoreMesh(core_axis_name="c", subcore_axis_name="s")
    fn = pl.kernel(
        _edge_body,
        out_type=jax.ShapeDtypeStruct((2, NPAD, 32), jnp.float32),
        mesh=mesh,
        scratch_types=[
            pltpu.VMEM((CPS, CH), jnp.int32),
            pltpu.VMEM((CPS, CH), jnp.int32),
            pltpu.VMEM((CPS, CH), jnp.float32),
            pltpu.VMEM((CH,), jnp.float32),
            pltpu.VMEM((NSLOT, CH, 32), jnp.float32),
            pltpu.VMEM((NSLOT, CH, 32), jnp.float32),
            pltpu.VMEM((32,), jnp.float32),
            pltpu.VMEM_SHARED((NPAD, 32), jnp.float32),
            pltpu.SemaphoreType.DMA((NSLOT,)),
            pltpu.SemaphoreType.DMA((NSLOT,)),
        ],
        compiler_params=pltpu.CompilerParams(use_tc_tiling_on_sc=False),
    )
    return fn(ps, pd, src2d, dst2d, ep2d, w0, zeros)


def kernel(node_h, node_pred, node_type_emb, edge_pred, node_graph_id,
           edge_index, jet_features, We0, be0, We1, be1, Wn0, bn0, Wn1, bn1,
           Wc0, bc0, Wc1, bc1, Wc2, bc2):
    gid3 = node_graph_id.reshape(NB, 1, BN)
    gidc = node_graph_id.reshape(N, 1)
    zeros_pt = jnp.zeros((ROWS_PT, 32), jnp.float32)
    rpad = RTOT * CH - E
    src2d = jnp.concatenate(
        [edge_index[0], jnp.zeros((rpad,), jnp.int32)]).reshape(RTOT, CH)
    dst2d = jnp.concatenate(
        [edge_index[1], jnp.full((rpad,), N, jnp.int32)]).reshape(RTOT, CH)
    ep2d = jnp.concatenate(
        [edge_pred, jnp.zeros((rpad,), jnp.float32)]).reshape(RTOT, CH)

    F, sums = _init_call(node_h, node_pred, node_type_emb, gid3)

    for We, be, Wn, bn in ((We0, be0, Wn0, bn0), (We1, be1, Wn1, bn1)):
        a2 = jnp.concatenate([We[33:34], We[72:77]], axis=0)
        c2 = jnp.concatenate([We[66:67], We[67:72]], axis=0)
        z1 = jnp.zeros((1, 32), jnp.float32)
        wfs = jnp.concatenate([We[1:33], a2, z1, z1], axis=0)
        wfd = jnp.concatenate([We[34:66], c2, be[None, :], z1], axis=0)
        wm = We[77:109]
        df = jnp.concatenate([Wn[0:32], Wn[69:70], Wn[64:69], bn[None, :], z1],
                             axis=0)
        d2 = Wn[32:64]
        ps, pd = _prep_call(F, sums, gidc, wfs, wfd, wm)
        aggp = _edge_stage(ps, pd, src2d, dst2d, ep2d, We[0], zeros_pt)
        F, sums = _upd_call(F, aggp, gid3, df, d2)

    return _fin_call(sums, jet_features, Wc0, bc0[None, :], Wc1, bc1[None, :],
                     Wc2, bc2[None, :])


# speedup vs baseline: 1.7355x; 1.7355x over previous
import functools

import jax
import jax.numpy as jnp
from jax import lax
from jax.experimental import pallas as pl
from jax.experimental.pallas import tpu as pltpu
from jax.experimental.pallas import tpu_sc as plsc

N = 50000
E = 800000
G = 512
H = 32

BN = 2000
NB = N // BN
F_DIM = 40

NPAD = 50176
ROWS_PT = NPAD // 16
CH = 128
CPT = 196
RTOT = 32 * CPT
NSLOT = 3
CPS = 14
NSUP = CPT // CPS


def _init_body(h_ref, p_ref, te_ref, gid_ref, f_ref, sums_ref):
    i = pl.program_id(0)
    h = h_ref[...]
    p = p_ref[...]
    te = te_ref[...]
    best = p[:, 0:1]
    am = jnp.zeros((BN, 1), jnp.float32)
    for j in range(1, 4):
        pj = p[:, j:j + 1]
        hit = pj > best
        best = jnp.where(hit, pj, best)
        am = jnp.where(hit, jnp.float32(j), am)
    ones = jnp.ones((BN, 1), jnp.float32)
    zeros = jnp.zeros((BN, 1), jnp.float32)
    F = jnp.concatenate([h, am, te, ones, zeros], axis=1)
    f_ref[...] = F
    gid = jnp.squeeze(gid_ref[...], 0)
    onehot_t = (gid == lax.broadcasted_iota(jnp.int32, (G, BN), 0))
    contrib = jnp.dot(onehot_t.astype(jnp.float32), F,
                      preferred_element_type=jnp.float32)

    @pl.when(i == 0)
    def _():
        sums_ref[...] = contrib

    @pl.when(i > 0)
    def _():
        sums_ref[...] += contrib


def _prep_body(f_ref, sums_ref, gidc_ref, wfs_ref, wfd_ref, wm_ref, pc_ref):
    F = f_ref[...]
    sums = sums_ref[...]
    mean = sums[:, :32] / jnp.maximum(sums[:, 38:39], 1.0)
    Mg = jnp.dot(mean, wm_ref[...], preferred_element_type=jnp.float32)
    gidc = gidc_ref[...]
    onehot = (gidc == lax.broadcasted_iota(jnp.int32, (BN, G), 1))
    ps = jnp.dot(F, wfs_ref[...], preferred_element_type=jnp.float32)
    pd = (jnp.dot(F, wfd_ref[...], preferred_element_type=jnp.float32)
          + jnp.dot(onehot.astype(jnp.float32), Mg,
                    preferred_element_type=jnp.float32))
    pc_ref[...] = jnp.stack([ps, pd])


def _upd_body(f_ref, agg_ref, gid_ref, df_ref, d2_ref, fn_ref, sums_ref):
    i = pl.program_id(0)
    F = f_ref[...]
    a = agg_ref[...]
    agg = a[0] + a[1]
    hn = jnp.maximum(
        jnp.dot(F, df_ref[...], preferred_element_type=jnp.float32)
        + jnp.dot(agg, d2_ref[...], preferred_element_type=jnp.float32), 0.0)
    Fn = jnp.concatenate([hn, F[:, 32:40]], axis=1)
    fn_ref[...] = Fn
    gid = jnp.squeeze(gid_ref[...], 0)
    onehot_t = (gid == lax.broadcasted_iota(jnp.int32, (G, BN), 0))
    contrib = jnp.dot(onehot_t.astype(jnp.float32), Fn,
                      preferred_element_type=jnp.float32)

    @pl.when(i == 0)
    def _():
        sums_ref[...] = contrib

    @pl.when(i > 0)
    def _():
        sums_ref[...] += contrib


def _fin_body(sums_ref, jet_ref, wc0_ref, bc0_ref, wc1_ref, bc1_ref,
              wc2_ref, bc2_ref, out_ref):
    sums = sums_ref[...]
    mean = sums[:, :32] / jnp.maximum(sums[:, 38:39], 1.0)
    gr = jnp.concatenate([mean, jet_ref[...]], axis=1)
    x = jnp.dot(gr, wc0_ref[...], preferred_element_type=jnp.float32) + bc0_ref[...]
    x = jnp.maximum(
        jnp.dot(x, wc1_ref[...], preferred_element_type=jnp.float32)
        + bc1_ref[...], 0.0)
    out_ref[...] = (jnp.dot(x, wc2_ref[...], preferred_element_type=jnp.float32)
                    + bc2_ref[...])


def _node_spec(w):
    return pl.BlockSpec((BN, w), lambda i: (i, 0))


def _full_spec(shape):
    nd = len(shape)
    return pl.BlockSpec(shape, lambda i: (0,) * nd)


def _init_call(node_h, node_pred, node_te, gid3):
    return pl.pallas_call(
        _init_body,
        grid=(NB,),
        in_specs=[_node_spec(32), _node_spec(4), _node_spec(5),
                  pl.BlockSpec((1, 1, BN), lambda i: (i, 0, 0))],
        out_specs=[_node_spec(F_DIM), _full_spec((G, F_DIM))],
        out_shape=[jax.ShapeDtypeStruct((N, F_DIM), jnp.float32),
                   jax.ShapeDtypeStruct((G, F_DIM), jnp.float32)],
    )(node_h, node_pred, node_te, gid3)


def _prep_call(F, sums, gidc, wfs, wfd, wm):
    return pl.pallas_call(
        _prep_body,
        grid=(NB,),
        in_specs=[_node_spec(F_DIM), _full_spec((G, F_DIM)), _node_spec(1),
                  _full_spec((F_DIM, 32)), _full_spec((F_DIM, 32)),
                  _full_spec((32, 32))],
        out_specs=pl.BlockSpec((2, BN, 32), lambda i: (0, i, 0)),
        out_shape=jax.ShapeDtypeStruct((2, N, 32), jnp.float32),
    )(F, sums, gidc, wfs, wfd, wm)


def _upd_call(F, aggp, gid3, df, d2):
    return pl.pallas_call(
        _upd_body,
        grid=(NB,),
        in_specs=[_node_spec(F_DIM),
                  pl.BlockSpec((2, BN, 32), lambda i: (0, i, 0)),
                  pl.BlockSpec((1, 1, BN), lambda i: (i, 0, 0)),
                  _full_spec((F_DIM, 32)), _full_spec((32, 32))],
        out_specs=[_node_spec(F_DIM), _full_spec((G, F_DIM))],
        out_shape=[jax.ShapeDtypeStruct((N, F_DIM), jnp.float32),
                   jax.ShapeDtypeStruct((G, F_DIM), jnp.float32)],
    )(F, aggp, gid3, df, d2)


def _fin_call(sums, jet, wc0, bc0, wc1, bc1, wc2, bc2):
    return pl.pallas_call(
        _fin_body,
        grid=(1,),
        in_specs=[_full_spec((G, F_DIM)), _full_spec((G, 10)),
                  _full_spec((42, 64)), _full_spec((1, 64)),
                  _full_spec((64, 64)), _full_spec((1, 64)),
                  _full_spec((64, 2)), _full_spec((1, 2))],
        out_specs=_full_spec((G, 2)),
        out_shape=jax.ShapeDtypeStruct((G, 2), jnp.float32),
    )(sums, jet, wc0, bc0, wc1, bc1, wc2, bc2)


def _edge_body(t_hbm, gi_hbm, di_hbm, ep_hbm, w0_hbm, zeros_hbm, out_hbm,
               gidxs, didxs, epv, tv, av, m0, m1, w0v, aggsh, gsem, s0, s1):
    c_ax = lax.axis_index("c")
    s_ax = lax.axis_index("s")
    pltpu.sync_copy(zeros_hbm, aggsh.at[pl.ds(s_ax * ROWS_PT, ROWS_PT)])
    pltpu.sync_copy(w0_hbm, w0v)
    plsc.subcore_barrier()
    tile = c_ax * 16 + s_ax

    def compute_chunk(c, mq):
        def sig_body(j, carry2):
            x = epv[c, pl.ds(j * 16, 16)]
            tv[pl.ds(j * 16, 16)] = 1.0 / (1.0 + jnp.exp(-x))
            return carry2

        lax.fori_loop(0, CH // 16, sig_body, 0)
        w0lo = w0v[pl.ds(0, 16)]
        w0hi = w0v[pl.ds(16, 16)]

        def group_body(g, carry3):
            tvec = tv[pl.ds(g * 16, 16)]
            for j in range(16):
                e = g * 16 + j
                t = tvec[j]
                x0 = av[e, pl.ds(0, 16)] + av[CH + e, pl.ds(0, 16)] + t * w0lo
                x1 = av[e, pl.ds(16, 16)] + av[CH + e, pl.ds(16, 16)] + t * w0hi
                mq[e, pl.ds(0, 16)] = 1.0 - 2.0 / (jnp.exp(x0 + x0) + 1.0)
                mq[e, pl.ds(16, 16)] = 1.0 - 2.0 / (jnp.exp(x1 + x1) + 1.0)
            return carry3

        lax.fori_loop(0, CH // 16, group_body, 0)

    def super_body(sb, carry0):
        r0 = tile * CPT + sb * CPS
        pltpu.sync_copy(gi_hbm.at[pl.ds(r0, CPS)], gidxs)
        pltpu.sync_copy(di_hbm.at[pl.ds(r0, CPS)], didxs)
        pltpu.sync_copy(ep_hbm.at[pl.ds(r0, CPS)], epv)

        def pair_body(p2, carry):
            for q, (mq, sq) in enumerate(((m0, s0), (m1, s1))):
                c = 2 * p2 + q
                pltpu.async_copy(t_hbm.at[gidxs.at[c]], av, gsem).wait()

                @pl.when(p2 > 0)
                def _():
                    pltpu.make_async_copy(mq, aggsh.at[didxs.at[c - 2]],
                                          sq).wait()

                compute_chunk(c, mq)
                pltpu.async_copy(mq, aggsh.at[didxs.at[c]], sq, add=True)
            return carry

        lax.fori_loop(0, CPS // 2, pair_body, 0)
        pltpu.make_async_copy(m0, aggsh.at[didxs.at[CPS - 2]], s0).wait()
        pltpu.make_async_copy(m1, aggsh.at[didxs.at[CPS - 1]], s1).wait()
        return carry0

    lax.fori_loop(0, NSUP, super_body, 0)
    plsc.subcore_barrier()
    pltpu.sync_copy(aggsh.at[pl.ds(s_ax * ROWS_PT, ROWS_PT)],
                    out_hbm.at[c_ax, pl.ds(s_ax * ROWS_PT, ROWS_PT)])


def _edge_stage(tcomb, gi2d, di2d, ep2d, w0, zeros):
    mesh = plsc.VectorSubcoreMesh(core_axis_name="c", subcore_axis_name="s")
    fn = pl.kernel(
        _edge_body,
        out_type=jax.ShapeDtypeStruct((2, NPAD, 32), jnp.float32),
        mesh=mesh,
        scratch_types=[
            pltpu.VMEM((CPS, 2 * CH), jnp.int32),
            pltpu.VMEM((CPS, CH), jnp.int32),
            pltpu.VMEM((CPS, CH), jnp.float32),
            pltpu.VMEM((CH,), jnp.float32),
            pltpu.VMEM((2 * CH, 32), jnp.float32),
            pltpu.VMEM((CH, 32), jnp.float32),
            pltpu.VMEM((CH, 32), jnp.float32),
            pltpu.VMEM((32,), jnp.float32),
            pltpu.VMEM_SHARED((NPAD, 32), jnp.float32),
            pltpu.SemaphoreType.DMA,
            pltpu.SemaphoreType.DMA,
            pltpu.SemaphoreType.DMA,
        ],
        compiler_params=pltpu.CompilerParams(use_tc_tiling_on_sc=False),
    )
    return fn(tcomb, gi2d, di2d, ep2d, w0, zeros)


def kernel(node_h, node_pred, node_type_emb, edge_pred, node_graph_id,
           edge_index, jet_features, We0, be0, We1, be1, Wn0, bn0, Wn1, bn1,
           Wc0, bc0, Wc1, bc1, Wc2, bc2):
    gid3 = node_graph_id.reshape(NB, 1, BN)
    gidc = node_graph_id.reshape(N, 1)
    zeros_pt = jnp.zeros((ROWS_PT, 32), jnp.float32)
    rpad = RTOT * CH - E
    src2d = jnp.concatenate(
        [edge_index[0], jnp.zeros((rpad,), jnp.int32)]).reshape(RTOT, CH)
    dstg2d = jnp.concatenate(
        [edge_index[1], jnp.zeros((rpad,), jnp.int32)]).reshape(RTOT, CH)
    gi2d = jnp.concatenate([src2d, dstg2d + N], axis=1)
    di2d = jnp.concatenate(
        [edge_index[1], jnp.full((rpad,), N, jnp.int32)]).reshape(RTOT, CH)
    ep2d = jnp.concatenate(
        [edge_pred, jnp.zeros((rpad,), jnp.float32)]).reshape(RTOT, CH)

    F, sums = _init_call(node_h, node_pred, node_type_emb, gid3)

    for We, be, Wn, bn in ((We0, be0, Wn0, bn0), (We1, be1, Wn1, bn1)):
        a2 = jnp.concatenate([We[33:34], We[72:77]], axis=0)
        c2 = jnp.concatenate([We[66:67], We[67:72]], axis=0)
        z1 = jnp.zeros((1, 32), jnp.float32)
        wfs = jnp.concatenate([We[1:33], a2, z1, z1], axis=0)
        wfd = jnp.concatenate([We[34:66], c2, be[None, :], z1], axis=0)
        wm = We[77:109]
        df = jnp.concatenate([Wn[0:32], Wn[69:70], Wn[64:69], bn[None, :], z1],
                             axis=0)
        d2 = Wn[32:64]
        pc = _prep_call(F, sums, gidc, wfs, wfd, wm)
        aggp = _edge_stage(pc.reshape(2 * N, 32), gi2d, di2d, ep2d, We[0],
                           zeros_pt)
        F, sums = _upd_call(F, aggp, gid3, df, d2)

    return _fin_call(sums, jet_features, Wc0, bc0[None, :], Wc1, bc1[None, :],
                     Wc2, bc2[None, :])


# speedup vs baseline: 2.1406x; 1.2334x over previous
import functools

import jax
import jax.numpy as jnp
from jax import lax
from jax.experimental import pallas as pl
from jax.experimental.pallas import tpu as pltpu
from jax.experimental.pallas import tpu_sc as plsc

N = 50000
E = 800000
G = 512
H = 32

BN = 2000
NB = N // BN
F_DIM = 40

NPAD = 50176
ROWS_PT = NPAD // 16
CH = 128
CPT = 196
RTOT = 32 * CPT
NSLOT = 3
CPS = 14
NSUP = CPT // CPS


def _init_body(h_ref, p_ref, te_ref, gid_ref, f_ref, sums_ref):
    i = pl.program_id(0)
    h = h_ref[...]
    p = p_ref[...]
    te = te_ref[...]
    best = p[:, 0:1]
    am = jnp.zeros((BN, 1), jnp.float32)
    for j in range(1, 4):
        pj = p[:, j:j + 1]
        hit = pj > best
        best = jnp.where(hit, pj, best)
        am = jnp.where(hit, jnp.float32(j), am)
    ones = jnp.ones((BN, 1), jnp.float32)
    zeros = jnp.zeros((BN, 1), jnp.float32)
    F = jnp.concatenate([h, am, te, ones, zeros], axis=1)
    f_ref[...] = F
    gid = jnp.squeeze(gid_ref[...], 0)
    onehot_t = (gid == lax.broadcasted_iota(jnp.int32, (G, BN), 0))
    contrib = jnp.dot(onehot_t.astype(jnp.float32), F,
                      preferred_element_type=jnp.float32)

    @pl.when(i == 0)
    def _():
        sums_ref[...] = contrib

    @pl.when(i > 0)
    def _():
        sums_ref[...] += contrib


def _prep_body(f_ref, sums_ref, gidc_ref, wfs_ref, wfd_ref, wm_ref, pc_ref):
    F = f_ref[...]
    sums = sums_ref[...]
    mean = sums[:, :32] / jnp.maximum(sums[:, 38:39], 1.0)
    Mg = jnp.dot(mean, wm_ref[...], preferred_element_type=jnp.float32)
    gidc = gidc_ref[...]
    onehot = (gidc == lax.broadcasted_iota(jnp.int32, (BN, G), 1))
    ps = jnp.dot(F, wfs_ref[...], preferred_element_type=jnp.float32)
    pd = (jnp.dot(F, wfd_ref[...], preferred_element_type=jnp.float32)
          + jnp.dot(onehot.astype(jnp.float32), Mg,
                    preferred_element_type=jnp.float32))
    pc_ref[...] = jnp.stack([ps, pd])


def _upd_body(f_ref, agg_ref, gid_ref, df_ref, d2_ref, fn_ref, sums_ref):
    i = pl.program_id(0)
    F = f_ref[...]
    a = agg_ref[...]
    agg = a[0] + a[1]
    hn = jnp.maximum(
        jnp.dot(F, df_ref[...], preferred_element_type=jnp.float32)
        + jnp.dot(agg, d2_ref[...], preferred_element_type=jnp.float32), 0.0)
    Fn = jnp.concatenate([hn, F[:, 32:40]], axis=1)
    fn_ref[...] = Fn
    gid = jnp.squeeze(gid_ref[...], 0)
    onehot_t = (gid == lax.broadcasted_iota(jnp.int32, (G, BN), 0))
    contrib = jnp.dot(onehot_t.astype(jnp.float32), Fn,
                      preferred_element_type=jnp.float32)

    @pl.when(i == 0)
    def _():
        sums_ref[...] = contrib

    @pl.when(i > 0)
    def _():
        sums_ref[...] += contrib


def _fin_body(sums_ref, jet_ref, wc0_ref, bc0_ref, wc1_ref, bc1_ref,
              wc2_ref, bc2_ref, out_ref):
    sums = sums_ref[...]
    mean = sums[:, :32] / jnp.maximum(sums[:, 38:39], 1.0)
    gr = jnp.concatenate([mean, jet_ref[...]], axis=1)
    x = jnp.dot(gr, wc0_ref[...], preferred_element_type=jnp.float32) + bc0_ref[...]
    x = jnp.maximum(
        jnp.dot(x, wc1_ref[...], preferred_element_type=jnp.float32)
        + bc1_ref[...], 0.0)
    out_ref[...] = (jnp.dot(x, wc2_ref[...], preferred_element_type=jnp.float32)
                    + bc2_ref[...])


def _node_spec(w):
    return pl.BlockSpec((BN, w), lambda i: (i, 0))


def _full_spec(shape):
    nd = len(shape)
    return pl.BlockSpec(shape, lambda i: (0,) * nd)


def _init_call(node_h, node_pred, node_te, gid3):
    return pl.pallas_call(
        _init_body,
        grid=(NB,),
        in_specs=[_node_spec(32), _node_spec(4), _node_spec(5),
                  pl.BlockSpec((1, 1, BN), lambda i: (i, 0, 0))],
        out_specs=[_node_spec(F_DIM), _full_spec((G, F_DIM))],
        out_shape=[jax.ShapeDtypeStruct((N, F_DIM), jnp.float32),
                   jax.ShapeDtypeStruct((G, F_DIM), jnp.float32)],
    )(node_h, node_pred, node_te, gid3)


def _prep_call(F, sums, gidc, wfs, wfd, wm):
    return pl.pallas_call(
        _prep_body,
        grid=(NB,),
        in_specs=[_node_spec(F_DIM), _full_spec((G, F_DIM)), _node_spec(1),
                  _full_spec((F_DIM, 32)), _full_spec((F_DIM, 32)),
                  _full_spec((32, 32))],
        out_specs=pl.BlockSpec((2, BN, 32), lambda i: (0, i, 0)),
        out_shape=jax.ShapeDtypeStruct((2, N, 32), jnp.float32),
    )(F, sums, gidc, wfs, wfd, wm)


def _upd_call(F, aggp, gid3, df, d2):
    return pl.pallas_call(
        _upd_body,
        grid=(NB,),
        in_specs=[_node_spec(F_DIM),
                  pl.BlockSpec((2, BN, 32), lambda i: (0, i, 0)),
                  pl.BlockSpec((1, 1, BN), lambda i: (i, 0, 0)),
                  _full_spec((F_DIM, 32)), _full_spec((32, 32))],
        out_specs=[_node_spec(F_DIM), _full_spec((G, F_DIM))],
        out_shape=[jax.ShapeDtypeStruct((N, F_DIM), jnp.float32),
                   jax.ShapeDtypeStruct((G, F_DIM), jnp.float32)],
    )(F, aggp, gid3, df, d2)


def _fin_call(sums, jet, wc0, bc0, wc1, bc1, wc2, bc2):
    return pl.pallas_call(
        _fin_body,
        grid=(1,),
        in_specs=[_full_spec((G, F_DIM)), _full_spec((G, 10)),
                  _full_spec((42, 64)), _full_spec((1, 64)),
                  _full_spec((64, 64)), _full_spec((1, 64)),
                  _full_spec((64, 2)), _full_spec((1, 2))],
        out_specs=_full_spec((G, 2)),
        out_shape=jax.ShapeDtypeStruct((G, 2), jnp.float32),
    )(sums, jet, wc0, bc0, wc1, bc1, wc2, bc2)


def _edge_body(t_hbm, gi_hbm, di_hbm, ep_hbm, w0_hbm, zeros_hbm, out_hbm,
               gidxs, didxs, epv, tv, av0, av1, w0v, aggsh, g0, g1, s0, s1):
    c_ax = lax.axis_index("c")
    s_ax = lax.axis_index("s")
    pltpu.sync_copy(zeros_hbm, aggsh.at[pl.ds(s_ax * ROWS_PT, ROWS_PT)])
    pltpu.sync_copy(w0_hbm, w0v)
    plsc.subcore_barrier()
    tile = c_ax * 16 + s_ax

    def compute_chunk(c, buf):
        def sig_body(j, carry2):
            x = epv[c, pl.ds(j * 16, 16)]
            tv[pl.ds(j * 16, 16)] = 1.0 / (1.0 + jnp.exp(-x))
            return carry2

        lax.fori_loop(0, CH // 16, sig_body, 0)
        w0lo = w0v[pl.ds(0, 16)]
        w0hi = w0v[pl.ds(16, 16)]

        def group_body(g, carry3):
            tvec = tv[pl.ds(g * 16, 16)]
            for j in range(16):
                e = g * 16 + j
                t = tvec[j]
                x0 = buf[e, pl.ds(0, 16)] + buf[CH + e, pl.ds(0, 16)] + t * w0lo
                x1 = buf[e, pl.ds(16, 16)] + buf[CH + e, pl.ds(16, 16)] + t * w0hi
                buf[e, pl.ds(0, 16)] = 1.0 - 2.0 / (jnp.exp(x0 + x0) + 1.0)
                buf[e, pl.ds(16, 16)] = 1.0 - 2.0 / (jnp.exp(x1 + x1) + 1.0)
            return carry3

        lax.fori_loop(0, CH // 16, group_body, 0)

    def super_body(sb, carry0):
        r0 = tile * CPT + sb * CPS
        pltpu.sync_copy(gi_hbm.at[pl.ds(r0, CPS)], gidxs)
        pltpu.sync_copy(di_hbm.at[pl.ds(r0, CPS)], didxs)
        pltpu.sync_copy(ep_hbm.at[pl.ds(r0, CPS)], epv)
        pltpu.async_copy(t_hbm.at[gidxs.at[0]], av0, g0)

        def pair_body(p2, carry):
            for q in (0, 1):
                buf, gq, sq = (av0, g0, s0) if q == 0 else (av1, g1, s1)
                obuf, ogq, osq = (av1, g1, s1) if q == 0 else (av0, g0, s0)
                c = 2 * p2 + q
                pltpu.make_async_copy(t_hbm.at[gidxs.at[c]], buf, gq).wait()

                @pl.when(c >= 1)
                def _():
                    pltpu.make_async_copy(obuf.at[pl.ds(0, CH)],
                                          aggsh.at[didxs.at[c - 1]],
                                          osq).wait()

                @pl.when(c + 1 < CPS)
                def _():
                    pltpu.async_copy(t_hbm.at[gidxs.at[c + 1]], obuf, ogq)

                compute_chunk(c, buf)
                pltpu.async_copy(buf.at[pl.ds(0, CH)],
                                 aggsh.at[didxs.at[c]], sq, add=True)
            return carry

        lax.fori_loop(0, CPS // 2, pair_body, 0)
        pltpu.make_async_copy(av1.at[pl.ds(0, CH)],
                              aggsh.at[didxs.at[CPS - 1]], s1).wait()
        return carry0

    lax.fori_loop(0, NSUP, super_body, 0)
    plsc.subcore_barrier()
    pltpu.sync_copy(aggsh.at[pl.ds(s_ax * ROWS_PT, ROWS_PT)],
                    out_hbm.at[c_ax, pl.ds(s_ax * ROWS_PT, ROWS_PT)])


def _edge_stage(tcomb, gi2d, di2d, ep2d, w0, zeros):
    mesh = plsc.VectorSubcoreMesh(core_axis_name="c", subcore_axis_name="s")
    fn = pl.kernel(
        _edge_body,
        out_type=jax.ShapeDtypeStruct((2, NPAD, 32), jnp.float32),
        mesh=mesh,
        scratch_types=[
            pltpu.VMEM((CPS, 2 * CH), jnp.int32),
            pltpu.VMEM((CPS, CH), jnp.int32),
            pltpu.VMEM((CPS, CH), jnp.float32),
            pltpu.VMEM((CH,), jnp.float32),
            pltpu.VMEM((2 * CH, 32), jnp.float32),
            pltpu.VMEM((2 * CH, 32), jnp.float32),
            pltpu.VMEM((32,), jnp.float32),
            pltpu.VMEM_SHARED((NPAD, 32), jnp.float32),
            pltpu.SemaphoreType.DMA,
            pltpu.SemaphoreType.DMA,
            pltpu.SemaphoreType.DMA,
            pltpu.SemaphoreType.DMA,
        ],
        compiler_params=pltpu.CompilerParams(use_tc_tiling_on_sc=False),
    )
    return fn(tcomb, gi2d, di2d, ep2d, w0, zeros)


def kernel(node_h, node_pred, node_type_emb, edge_pred, node_graph_id,
           edge_index, jet_features, We0, be0, We1, be1, Wn0, bn0, Wn1, bn1,
           Wc0, bc0, Wc1, bc1, Wc2, bc2):
    gid3 = node_graph_id.reshape(NB, 1, BN)
    gidc = node_graph_id.reshape(N, 1)
    zeros_pt = jnp.zeros((ROWS_PT, 32), jnp.float32)
    rpad = RTOT * CH - E
    src2d = jnp.concatenate(
        [edge_index[0], jnp.zeros((rpad,), jnp.int32)]).reshape(RTOT, CH)
    dstg2d = jnp.concatenate(
        [edge_index[1], jnp.zeros((rpad,), jnp.int32)]).reshape(RTOT, CH)
    gi2d = jnp.concatenate([src2d, dstg2d + N], axis=1)
    di2d = jnp.concatenate(
        [edge_index[1], jnp.full((rpad,), N, jnp.int32)]).reshape(RTOT, CH)
    ep2d = jnp.concatenate(
        [edge_pred, jnp.zeros((rpad,), jnp.float32)]).reshape(RTOT, CH)

    F, sums = _init_call(node_h, node_pred, node_type_emb, gid3)

    for We, be, Wn, bn in ((We0, be0, Wn0, bn0), (We1, be1, Wn1, bn1)):
        a2 = jnp.concatenate([We[33:34], We[72:77]], axis=0)
        c2 = jnp.concatenate([We[66:67], We[67:72]], axis=0)
        z1 = jnp.zeros((1, 32), jnp.float32)
        wfs = jnp.concatenate([We[1:33], a2, z1, z1], axis=0)
        wfd = jnp.concatenate([We[34:66], c2, be[None, :], z1], axis=0)
        wm = We[77:109]
        df = jnp.concatenate([Wn[0:32], Wn[69:70], Wn[64:69], bn[None, :], z1],
                             axis=0)
        d2 = Wn[32:64]
        pc = _prep_call(F, sums, gidc, wfs, wfd, wm)
        aggp = _edge_stage(pc.reshape(2 * N, 32), gi2d, di2d, ep2d, We[0],
                           zeros_pt)
        F, sums = _upd_call(F, aggp, gid3, df, d2)

    return _fin_call(sums, jet_features, Wc0, bc0[None, :], Wc1, bc1[None, :],
                     Wc2, bc2[None, :])


# speedup vs baseline: 2.2043x; 1.0297x over previous
import functools

import jax
import jax.numpy as jnp
from jax import lax
from jax.experimental import pallas as pl
from jax.experimental.pallas import tpu as pltpu
from jax.experimental.pallas import tpu_sc as plsc

N = 50000
E = 800000
G = 512
H = 32

BN = 2000
NB = N // BN
F_DIM = 40

NPAD = 50176
ROWS_PT = NPAD // 16
CH = 128
CPT = 196
RTOT = 32 * CPT
NSLOT = 3
CPS = 14
NSUP = CPT // CPS


def _init_body(h_ref, p_ref, te_ref, gid_ref, f_ref, sums_ref):
    i = pl.program_id(0)
    h = h_ref[...]
    p = p_ref[...]
    te = te_ref[...]
    best = p[:, 0:1]
    am = jnp.zeros((BN, 1), jnp.float32)
    for j in range(1, 4):
        pj = p[:, j:j + 1]
        hit = pj > best
        best = jnp.where(hit, pj, best)
        am = jnp.where(hit, jnp.float32(j), am)
    ones = jnp.ones((BN, 1), jnp.float32)
    zeros = jnp.zeros((BN, 1), jnp.float32)
    F = jnp.concatenate([h, am, te, ones, zeros], axis=1)
    f_ref[...] = F
    gid = jnp.squeeze(gid_ref[...], 0)
    onehot_t = (gid == lax.broadcasted_iota(jnp.int32, (G, BN), 0))
    contrib = jnp.dot(onehot_t.astype(jnp.float32), F,
                      preferred_element_type=jnp.float32)

    @pl.when(i == 0)
    def _():
        sums_ref[...] = contrib

    @pl.when(i > 0)
    def _():
        sums_ref[...] += contrib


def _prep_body(f_ref, sums_ref, gidc_ref, wfs_ref, wfd_ref, wm_ref, pc_ref):
    F = f_ref[...]
    sums = sums_ref[...]
    mean = sums[:, :32] / jnp.maximum(sums[:, 38:39], 1.0)
    Mg = jnp.dot(mean, wm_ref[...], preferred_element_type=jnp.float32)
    gidc = gidc_ref[...]
    onehot = (gidc == lax.broadcasted_iota(jnp.int32, (BN, G), 1))
    ps = jnp.dot(F, wfs_ref[...], preferred_element_type=jnp.float32)
    pd = (jnp.dot(F, wfd_ref[...], preferred_element_type=jnp.float32)
          + jnp.dot(onehot.astype(jnp.float32), Mg,
                    preferred_element_type=jnp.float32))
    pc_ref[...] = jnp.stack([ps, pd])


def _upd_body(f_ref, agg_ref, gid_ref, df_ref, d2_ref, fn_ref, sums_ref):
    i = pl.program_id(0)
    F = f_ref[...]
    a = agg_ref[...]
    agg = a[0] + a[1]
    hn = jnp.maximum(
        jnp.dot(F, df_ref[...], preferred_element_type=jnp.float32)
        + jnp.dot(agg, d2_ref[...], preferred_element_type=jnp.float32), 0.0)
    Fn = jnp.concatenate([hn, F[:, 32:40]], axis=1)
    fn_ref[...] = Fn
    gid = jnp.squeeze(gid_ref[...], 0)
    onehot_t = (gid == lax.broadcasted_iota(jnp.int32, (G, BN), 0))
    contrib = jnp.dot(onehot_t.astype(jnp.float32), Fn,
                      preferred_element_type=jnp.float32)

    @pl.when(i == 0)
    def _():
        sums_ref[...] = contrib

    @pl.when(i > 0)
    def _():
        sums_ref[...] += contrib


def _fin_body(sums_ref, jet_ref, wc0_ref, bc0_ref, wc1_ref, bc1_ref,
              wc2_ref, bc2_ref, out_ref):
    sums = sums_ref[...]
    mean = sums[:, :32] / jnp.maximum(sums[:, 38:39], 1.0)
    gr = jnp.concatenate([mean, jet_ref[...]], axis=1)
    x = jnp.dot(gr, wc0_ref[...], preferred_element_type=jnp.float32) + bc0_ref[...]
    x = jnp.maximum(
        jnp.dot(x, wc1_ref[...], preferred_element_type=jnp.float32)
        + bc1_ref[...], 0.0)
    out_ref[...] = (jnp.dot(x, wc2_ref[...], preferred_element_type=jnp.float32)
                    + bc2_ref[...])


def _node_spec(w):
    return pl.BlockSpec((BN, w), lambda i: (i, 0))


def _full_spec(shape):
    nd = len(shape)
    return pl.BlockSpec(shape, lambda i: (0,) * nd)


def _init_call(node_h, node_pred, node_te, gid3):
    return pl.pallas_call(
        _init_body,
        grid=(NB,),
        in_specs=[_node_spec(32), _node_spec(4), _node_spec(5),
                  pl.BlockSpec((1, 1, BN), lambda i: (i, 0, 0))],
        out_specs=[_node_spec(F_DIM), _full_spec((G, F_DIM))],
        out_shape=[jax.ShapeDtypeStruct((N, F_DIM), jnp.float32),
                   jax.ShapeDtypeStruct((G, F_DIM), jnp.float32)],
    )(node_h, node_pred, node_te, gid3)


def _prep_call(F, sums, gidc, wfs, wfd, wm):
    return pl.pallas_call(
        _prep_body,
        grid=(NB,),
        in_specs=[_node_spec(F_DIM), _full_spec((G, F_DIM)), _node_spec(1),
                  _full_spec((F_DIM, 32)), _full_spec((F_DIM, 32)),
                  _full_spec((32, 32))],
        out_specs=pl.BlockSpec((2, BN, 32), lambda i: (0, i, 0)),
        out_shape=jax.ShapeDtypeStruct((2, N, 32), jnp.float32),
    )(F, sums, gidc, wfs, wfd, wm)


def _upd_call(F, aggp, gid3, df, d2):
    return pl.pallas_call(
        _upd_body,
        grid=(NB,),
        in_specs=[_node_spec(F_DIM),
                  pl.BlockSpec((2, BN, 32), lambda i: (0, i, 0)),
                  pl.BlockSpec((1, 1, BN), lambda i: (i, 0, 0)),
                  _full_spec((F_DIM, 32)), _full_spec((32, 32))],
        out_specs=[_node_spec(F_DIM), _full_spec((G, F_DIM))],
        out_shape=[jax.ShapeDtypeStruct((N, F_DIM), jnp.float32),
                   jax.ShapeDtypeStruct((G, F_DIM), jnp.float32)],
    )(F, aggp, gid3, df, d2)


def _fin_call(sums, jet, wc0, bc0, wc1, bc1, wc2, bc2):
    return pl.pallas_call(
        _fin_body,
        grid=(1,),
        in_specs=[_full_spec((G, F_DIM)), _full_spec((G, 10)),
                  _full_spec((42, 64)), _full_spec((1, 64)),
                  _full_spec((64, 64)), _full_spec((1, 64)),
                  _full_spec((64, 2)), _full_spec((1, 2))],
        out_specs=_full_spec((G, 2)),
        out_shape=jax.ShapeDtypeStruct((G, 2), jnp.float32),
    )(sums, jet, wc0, bc0, wc1, bc1, wc2, bc2)


def _edge_body(t_hbm, gi_hbm, di_hbm, ep_hbm, w0_hbm, zeros_hbm, out_hbm,
               gidxs, didxs, epv, tv, av0, av1, w0v, aggsh, g0, g1, s0, s1):
    c_ax = lax.axis_index("c")
    s_ax = lax.axis_index("s")
    pltpu.sync_copy(zeros_hbm, aggsh.at[pl.ds(s_ax * ROWS_PT, ROWS_PT)])
    pltpu.sync_copy(w0_hbm, w0v)
    plsc.subcore_barrier()
    tile = c_ax * 16 + s_ax

    def compute_chunk(c, buf):
        w0lo = w0v[pl.ds(0, 16)]
        w0hi = w0v[pl.ds(16, 16)]

        @plsc.parallel_loop(0, CH // 16, 1)
        def group_body(g):
            x = epv[c, pl.ds(g * 16, 16)]
            tvec = 1.0 / (1.0 + jnp.exp(-x))
            for j in range(16):
                e = g * 16 + j
                t = tvec[j]
                x0 = buf[e, pl.ds(0, 16)] + buf[CH + e, pl.ds(0, 16)] + t * w0lo
                x1 = buf[e, pl.ds(16, 16)] + buf[CH + e, pl.ds(16, 16)] + t * w0hi
                buf[e, pl.ds(0, 16)] = 1.0 - 2.0 / (jnp.exp(x0 + x0) + 1.0)
                buf[e, pl.ds(16, 16)] = 1.0 - 2.0 / (jnp.exp(x1 + x1) + 1.0)

    def super_body(sb, carry0):
        r0 = tile * CPT + sb * CPS
        pltpu.sync_copy(gi_hbm.at[pl.ds(r0, CPS)], gidxs)
        pltpu.sync_copy(di_hbm.at[pl.ds(r0, CPS)], didxs)
        pltpu.sync_copy(ep_hbm.at[pl.ds(r0, CPS)], epv)
        pltpu.async_copy(t_hbm.at[gidxs.at[0]], av0, g0)

        def pair_body(p2, carry):
            for q in (0, 1):
                buf, gq, sq = (av0, g0, s0) if q == 0 else (av1, g1, s1)
                obuf, ogq, osq = (av1, g1, s1) if q == 0 else (av0, g0, s0)
                c = 2 * p2 + q
                pltpu.make_async_copy(t_hbm.at[gidxs.at[c]], buf, gq).wait()

                @pl.when(c >= 1)
                def _():
                    pltpu.make_async_copy(obuf.at[pl.ds(0, CH)],
                                          aggsh.at[didxs.at[c - 1]],
                                          osq).wait()

                @pl.when(c + 1 < CPS)
                def _():
                    pltpu.async_copy(t_hbm.at[gidxs.at[c + 1]], obuf, ogq)

                compute_chunk(c, buf)
                pltpu.async_copy(buf.at[pl.ds(0, CH)],
                                 aggsh.at[didxs.at[c]], sq, add=True)
            return carry

        lax.fori_loop(0, CPS // 2, pair_body, 0)
        pltpu.make_async_copy(av1.at[pl.ds(0, CH)],
                              aggsh.at[didxs.at[CPS - 1]], s1).wait()
        return carry0

    lax.fori_loop(0, NSUP, super_body, 0)
    plsc.subcore_barrier()
    pltpu.sync_copy(aggsh.at[pl.ds(s_ax * ROWS_PT, ROWS_PT)],
                    out_hbm.at[c_ax, pl.ds(s_ax * ROWS_PT, ROWS_PT)])


def _edge_stage(tcomb, gi2d, di2d, ep2d, w0, zeros):
    mesh = plsc.VectorSubcoreMesh(core_axis_name="c", subcore_axis_name="s")
    fn = pl.kernel(
        _edge_body,
        out_type=jax.ShapeDtypeStruct((2, NPAD, 32), jnp.float32),
        mesh=mesh,
        scratch_types=[
            pltpu.VMEM((CPS, 2 * CH), jnp.int32),
            pltpu.VMEM((CPS, CH), jnp.int32),
            pltpu.VMEM((CPS, CH), jnp.float32),
            pltpu.VMEM((CH,), jnp.float32),
            pltpu.VMEM((2 * CH, 32), jnp.float32),
            pltpu.VMEM((2 * CH, 32), jnp.float32),
            pltpu.VMEM((32,), jnp.float32),
            pltpu.VMEM_SHARED((NPAD, 32), jnp.float32),
            pltpu.SemaphoreType.DMA,
            pltpu.SemaphoreType.DMA,
            pltpu.SemaphoreType.DMA,
            pltpu.SemaphoreType.DMA,
        ],
        compiler_params=pltpu.CompilerParams(use_tc_tiling_on_sc=False),
    )
    return fn(tcomb, gi2d, di2d, ep2d, w0, zeros)


def kernel(node_h, node_pred, node_type_emb, edge_pred, node_graph_id,
           edge_index, jet_features, We0, be0, We1, be1, Wn0, bn0, Wn1, bn1,
           Wc0, bc0, Wc1, bc1, Wc2, bc2):
    gid3 = node_graph_id.reshape(NB, 1, BN)
    gidc = node_graph_id.reshape(N, 1)
    zeros_pt = jnp.zeros((ROWS_PT, 32), jnp.float32)
    rpad = RTOT * CH - E
    src2d = jnp.concatenate(
        [edge_index[0], jnp.zeros((rpad,), jnp.int32)]).reshape(RTOT, CH)
    dstg2d = jnp.concatenate(
        [edge_index[1], jnp.zeros((rpad,), jnp.int32)]).reshape(RTOT, CH)
    gi2d = jnp.concatenate([src2d, dstg2d + N], axis=1)
    di2d = jnp.concatenate(
        [edge_index[1], jnp.full((rpad,), N, jnp.int32)]).reshape(RTOT, CH)
    ep2d = jnp.concatenate(
        [edge_pred, jnp.zeros((rpad,), jnp.float32)]).reshape(RTOT, CH)

    F, sums = _init_call(node_h, node_pred, node_type_emb, gid3)

    for We, be, Wn, bn in ((We0, be0, Wn0, bn0), (We1, be1, Wn1, bn1)):
        a2 = jnp.concatenate([We[33:34], We[72:77]], axis=0)
        c2 = jnp.concatenate([We[66:67], We[67:72]], axis=0)
        z1 = jnp.zeros((1, 32), jnp.float32)
        wfs = jnp.concatenate([We[1:33], a2, z1, z1], axis=0)
        wfd = jnp.concatenate([We[34:66], c2, be[None, :], z1], axis=0)
        wm = We[77:109]
        df = jnp.concatenate([Wn[0:32], Wn[69:70], Wn[64:69], bn[None, :], z1],
                             axis=0)
        d2 = Wn[32:64]
        pc = _prep_call(F, sums, gidc, wfs, wfd, wm)
        aggp = _edge_stage(pc.reshape(2 * N, 32), gi2d, di2d, ep2d, We[0],
                           zeros_pt)
        F, sums = _upd_call(F, aggp, gid3, df, d2)

    return _fin_call(sums, jet_features, Wc0, bc0[None, :], Wc1, bc1[None, :],
                     Wc2, bc2[None, :])
